# baseline (speedup 1.0000x reference)
import functools
import math

import jax
import jax.numpy as jnp
from jax import lax
from jax.experimental import pallas as pl
from jax.experimental.pallas import tpu as pltpu
from jax.experimental.pallas import tpu_sc as plsc

HID = 128
N_NODES_C = 10000
N_EDGES_C = 160000

NC, NS = 2, 16
NW = NC * NS
CH = 128
EPW = 5120
E_PAD = NW * EPW
NCH = EPW // CH
N_PAD = 10112
RPT = N_PAD // NS
DUMMY = N_NODES_C
DW = 128

ETILE = 2048
NTILE = 1264

_SQRT2 = math.sqrt(2.0)


def _dot(a, b):
    return lax.dot_general(a.astype(jnp.bfloat16), b.astype(jnp.bfloat16),
                           (((1,), (0,)), ((), ())),
                           preferred_element_type=jnp.float32)


def _gelu(x):
    return 0.5 * x * (1.0 + lax.erf(x / _SQRT2))


def _layernorm(x, g, b):
    m = jnp.mean(x, axis=1, keepdims=True)
    v = jnp.mean((x - m) * (x - m), axis=1, keepdims=True)
    return (x - m) * lax.rsqrt(v + 1e-5) * g + b


def _row2(v):
    return v.reshape(1, -1)


def _mlp2_body(x_ref, w1, b1, w2, b2, o_ref):
    h = _gelu(_dot(x_ref[...], w1[...]) + b1[...])
    o_ref[...] = _dot(h, w2[...]) + b2[...]


def _mlp2_call(x, p, tile):
    n, din = x.shape
    dh = p[0].shape[1]
    dout = p[2].shape[1]
    grid = (n // tile,)
    return pl.pallas_call(
        _mlp2_body,
        grid=grid,
        in_specs=[
            pl.BlockSpec((tile, din), lambda i: (i, 0)),
            pl.BlockSpec((din, dh), lambda i: (0, 0)),
            pl.BlockSpec((1, dh), lambda i: (0, 0)),
            pl.BlockSpec((dh, dout), lambda i: (0, 0)),
            pl.BlockSpec((1, dout), lambda i: (0, 0)),
        ],
        out_specs=pl.BlockSpec((tile, dout), lambda i: (i, 0)),
        out_shape=jax.ShapeDtypeStruct((n, dout), jnp.float32),
    )(x, p[0], _row2(p[1]), p[2], _row2(p[3]))


def _edge_block_body(sf_ref, df_ref, ef_ref, w1s, w1d, w1e, b1,
                     w2m, b2m, w2e, b2e, g_ref, bb_ref,
                     msg_ref, efo_ref):
    h = (_dot(sf_ref[...], w1s[...]) + _dot(df_ref[...], w1d[...])
         + _dot(ef_ref[...], w1e[...]) + b1[...])
    h = _gelu(h)
    msg_ref[...] = _dot(h[:, :HID], w2m[...]) + b2m[...]
    eu = _dot(h[:, HID:], w2e[...]) + b2e[...]
    efo_ref[...] = _layernorm(ef_ref[...] + eu, g_ref[...], bb_ref[...])


def _edge_block_call(sf, df, ef, wp):
    grid = (E_PAD // ETILE,)
    row = pl.BlockSpec((ETILE, HID), lambda i: (i, 0))
    cst = lambda s: pl.BlockSpec(s, lambda i: (0, 0))
    return pl.pallas_call(
        _edge_block_body,
        grid=grid,
        in_specs=[row, row, row,
                  cst((HID, 2 * HID)), cst((HID, 2 * HID)), cst((HID, 2 * HID)),
                  cst((1, 2 * HID)),
                  cst((HID, HID)), cst((1, HID)),
                  cst((HID, HID)), cst((1, HID)),
                  cst((1, HID)), cst((1, HID))],
        out_specs=[row, row],
        out_shape=[jax.ShapeDtypeStruct((E_PAD, HID), jnp.float32),
                   jax.ShapeDtypeStruct((E_PAD, HID), jnp.float32)],
    )(sf, df, ef, *wp)


def _node_block_body(nf_ref, a0, a1, d0, d1, w1a, w1b, b1, w2, b2, g_ref, bb_ref,
                     out_ref):
    deg = d0[:, :1] + d1[:, :1]
    agg = (a0[...] + a1[...]) / (deg + 1e-8)
    h = _gelu(_dot(nf_ref[...], w1a[...]) + _dot(agg, w1b[...]) + b1[...])
    nu = _dot(h, w2[...]) + b2[...]
    out_ref[...] = _layernorm(nf_ref[...] + nu, g_ref[...], bb_ref[...])


def _node_block_call(nf, a0, a1, d0, d1, wp):
    grid = (N_PAD // NTILE,)
    row = pl.BlockSpec((NTILE, HID), lambda i: (i, 0))
    drow = pl.BlockSpec((NTILE, DW), lambda i: (i, 0))
    cst = lambda s: pl.BlockSpec(s, lambda i: (0, 0))
    return pl.pallas_call(
        _node_block_body,
        grid=grid,
        in_specs=[row, row, row, drow, drow,
                  cst((HID, HID)), cst((HID, HID)), cst((1, HID)),
                  cst((HID, HID)), cst((1, HID)),
                  cst((1, HID)), cst((1, HID))],
        out_specs=row,
        out_shape=jax.ShapeDtypeStruct((N_PAD, HID), jnp.float32),
    )(nf, a0, a1, d0, d1, *wp)


def _softplus(x):
    return jnp.maximum(x, 0.0) + jnp.log(1.0 + jnp.exp(-jnp.abs(x)))


def _tail_body(nf_ref, wp, bp,
               w1_fe, b1_fe, w2_fe, b2_fe,
               w1_st, b1_st, w2_st, b2_st,
               w1_gp, b1_gp, w2_gp, b2_gp,
               w1_co, b1_co, w2_co, b2_co,
               w1_so, b1_so, w2_so, b2_so,
               o_fe, o_st, o_gp, o_co, o_so):
    pooled = _gelu(_dot(nf_ref[...], wp[...]) + bp[...])
    ridx = lax.broadcasted_iota(jnp.int32, (N_PAD, HID), 0)
    valid = ridx < N_NODES_C
    psum = jnp.sum(jnp.where(valid, pooled, 0.0), axis=0, keepdims=True)
    pmean = psum / float(N_NODES_C)
    pmax = jnp.max(jnp.where(valid, pooled, -1e30), axis=0, keepdims=True)
    g = jnp.concatenate([pmean, pmax], axis=1)

    def head(w1, b1, w2, b2):
        h = _gelu(_dot(g, w1[...]) + b1[...])
        return _dot(h, w2[...]) + b2[...]

    o_fe[...] = head(w1_fe, b1_fe, w2_fe, b2_fe)
    o_st[...] = jax.nn.sigmoid(head(w1_st, b1_st, w2_st, b2_st))
    o_gp[...] = _softplus(head(w1_gp, b1_gp, w2_gp, b2_gp))
    o_co[...] = head(w1_co, b1_co, w2_co, b2_co)
    o_so[...] = head(w1_so, b1_so, w2_so, b2_so)


def _tail_call(nf, pool, heads):
    cst = lambda s: pl.BlockSpec(s, lambda: (0, 0))
    args = [nf, pool[0], _row2(pool[1])]
    in_specs = [cst((N_PAD, HID)), cst((HID, HID)), cst((1, HID))]
    for name in ('fe', 'stab', 'gap', 'coord', 'solv'):
        w1, b1, w2, b2 = heads[name]
        k = w2.shape[1]
        w2p = jnp.pad(w2, ((0, 0), (0, 8 - k)))
        b2p = jnp.pad(b2, (0, 8 - k))
        args += [w1, _row2(b1), w2p, _row2(b2p)]
        in_specs += [cst((2 * HID, HID)), cst((1, HID)), cst((HID, 8)), cst((1, 8))]
    out_specs = [cst((1, 8))] * 5
    out_shape = [jax.ShapeDtypeStruct((1, 8), jnp.float32)] * 5
    return pl.pallas_call(
        _tail_body,
        in_specs=in_specs,
        out_specs=out_specs,
        out_shape=out_shape,
    )(*args)


def _sc_mesh():
    return plsc.VectorSubcoreMesh(core_axis_name="c", subcore_axis_name="s",
                                  num_cores=NC, num_subcores=NS)


@functools.cache
def _build_gather_kernel():
    return functools.partial(
        pl.kernel,
        out_type=(jax.ShapeDtypeStruct((E_PAD, HID), jnp.float32),
                  jax.ShapeDtypeStruct((E_PAD, HID), jnp.float32)),
        mesh=_sc_mesh(),
        scratch_types=[pltpu.VMEM((CH,), jnp.int32), pltpu.VMEM((CH,), jnp.int32),
                       pltpu.VMEM((CH, HID), jnp.float32),
                       pltpu.VMEM((CH, HID), jnp.float32),
                       pltpu.SemaphoreType.DMA, pltpu.SemaphoreType.DMA],
    )(_gather_body)


def _gather_body(nf_hbm, src_hbm, dst_hbm, sf_hbm, df_hbm,
                 si_v, di_v, sr_v, dr_v, sem_s, sem_d):
    cid = lax.axis_index("c")
    sid = lax.axis_index("s")
    wbase = (cid * NS + sid) * EPW

    def body(c, carry):
        base = pl.multiple_of(wbase + c * CH, CH)
        pltpu.sync_copy(src_hbm.at[pl.ds(base, CH)], si_v)
        pltpu.sync_copy(dst_hbm.at[pl.ds(base, CH)], di_v)
        g1 = pltpu.async_copy(nf_hbm.at[si_v], sr_v, sem_s)
        g2 = pltpu.async_copy(nf_hbm.at[di_v], dr_v, sem_d)
        g1.wait()
        g2.wait()
        pltpu.sync_copy(sr_v, sf_hbm.at[pl.ds(base, CH)])
        pltpu.sync_copy(dr_v, df_hbm.at[pl.ds(base, CH)])
        return carry

    lax.fori_loop(0, NCH, body, 0)


@functools.cache
def _build_scatter_kernel():
    return functools.partial(
        pl.kernel,
        out_type=jax.ShapeDtypeStruct((NC, N_PAD, HID), jnp.float32),
        mesh=_sc_mesh(),
        scratch_types=[pltpu.VMEM((CH,), jnp.int32),
                       pltpu.VMEM((CH, HID), jnp.float32),
                       pltpu.VMEM_SHARED((N_PAD, HID), jnp.float32)],
    )(_scatter_body)


def _scatter_body(msg_hbm, dst_hbm, zeros_hbm, out_hbm, idx_v, rows_v, acc_sh):
    cid = lax.axis_index("c")
    sid = lax.axis_index("s")
    slab = pl.ds(sid * RPT, RPT)
    pltpu.sync_copy(zeros_hbm, acc_sh.at[slab])
    plsc.subcore_barrier()
    wbase = (cid * NS + sid) * EPW

    def body(c, carry):
        base = pl.multiple_of(wbase + c * CH, CH)
        pltpu.sync_copy(dst_hbm.at[pl.ds(base, CH)], idx_v)
        pltpu.sync_copy(msg_hbm.at[pl.ds(base, CH)], rows_v)
        pltpu.sync_copy(rows_v, acc_sh.at[idx_v], add=True)
        return carry

    lax.fori_loop(0, NCH, body, 0)
    plsc.subcore_barrier()
    pltpu.sync_copy(acc_sh.at[slab], out_hbm.at[cid].at[slab])


@functools.cache
def _build_degree_kernel():
    return functools.partial(
        pl.kernel,
        out_type=jax.ShapeDtypeStruct((NC, N_PAD, DW), jnp.float32),
        mesh=_sc_mesh(),
        scratch_types=[pltpu.VMEM((CH,), jnp.int32),
                       pltpu.VMEM((CH, DW), jnp.float32),
                       pltpu.VMEM_SHARED((N_PAD, DW), jnp.float32)],
    )(_degree_body)


def _degree_body(dst_hbm, ones_hbm, zeros_hbm, out_hbm, idx_v, ones_v, acc_sh):
    cid = lax.axis_index("c")
    sid = lax.axis_index("s")
    slab = pl.ds(sid * RPT, RPT)
    pltpu.sync_copy(zeros_hbm, acc_sh.at[slab])
    pltpu.sync_copy(ones_hbm, ones_v)
    plsc.subcore_barrier()
    wbase = (cid * NS + sid) * EPW

    def body(c, carry):
        base = pl.multiple_of(wbase + c * CH, CH)
        pltpu.sync_copy(dst_hbm.at[pl.ds(base, CH)], idx_v)
        pltpu.sync_copy(ones_v, acc_sh.at[idx_v], add=True)
        return carry

    lax.fori_loop(0, NCH, body, 0)
    plsc.subcore_barrier()
    pltpu.sync_copy(acc_sh.at[slab], out_hbm.at[cid].at[slab])


def _edge_weights(blk):
    w1m, b1m, w2m, b2m = blk['msg']
    w1e, b1e, w2e, b2e = blk['edge_upd']
    w1s = jnp.concatenate([w1m[:HID], w1e[:HID]], axis=1)
    w1d = jnp.concatenate([w1m[HID:2 * HID], w1e[HID:2 * HID]], axis=1)
    w1f = jnp.concatenate([w1m[2 * HID:], w1e[2 * HID:]], axis=1)
    b1 = _row2(jnp.concatenate([b1m, b1e]))
    return (w1s, w1d, w1f, b1, w2m, _row2(b2m), w2e, _row2(b2e),
            _row2(blk['en_g']), _row2(blk['en_b']))


def _node_weights(blk):
    w1, b1, w2, b2 = blk['node_upd']
    return (w1[:HID], w1[HID:], _row2(b1), w2, _row2(b2),
            _row2(blk['nn_g']), _row2(blk['nn_b']))


def kernel(node_feat_raw, edge_feat_raw, edge_index, params):
    n_nodes = node_feat_raw.shape[0]
    n_edges = edge_feat_raw.shape[0]

    nfr = jnp.pad(node_feat_raw, ((0, N_PAD - n_nodes), (0, 0)))
    efr = jnp.pad(edge_feat_raw, ((0, E_PAD - n_edges), (0, 0)))
    src = jnp.pad(edge_index[0], (0, E_PAD - n_edges))
    dst = jnp.pad(edge_index[1], (0, E_PAD - n_edges),
                  constant_values=DUMMY)

    zeros_big = jnp.zeros((RPT, HID), jnp.float32)
    zeros_deg = jnp.zeros((RPT, DW), jnp.float32)
    ones_deg = jnp.ones((CH, DW), jnp.float32)

    nf = _mlp2_call(nfr, params['node_emb'], NTILE)
    ef = _mlp2_call(efr, params['edge_emb'], ETILE)

    degs = _build_degree_kernel()(dst, ones_deg, zeros_deg)
    d0 = degs[0]
    d1 = degs[1]

    for blk in params['blocks']:
        sf, df = _build_gather_kernel()(nf, src, dst)
        msg, ef = _edge_block_call(sf, df, ef, _edge_weights(blk))
        aggs = _build_scatter_kernel()(msg, dst, zeros_big)
        nf = _node_block_call(nf, aggs[0], aggs[1], d0, d1, _node_weights(blk))

    o_fe, o_st, o_gp, o_co, o_so = _tail_call(nf, params['pool'], params['heads'])
    return jnp.concatenate([o_fe[0, :1], o_st[0, :1], o_gp[0, :1],
                            o_co[0, :4], o_so[0, :3]])


# speedup vs baseline: 1.1554x; 1.1554x over previous
import functools
import math

import jax
import jax.numpy as jnp
from jax import lax
from jax.experimental import pallas as pl
from jax.experimental.pallas import tpu as pltpu
from jax.experimental.pallas import tpu_sc as plsc

HID = 128
N_NODES_C = 10000
N_EDGES_C = 160000

NC, NS = 2, 16
NW = NC * NS
CH = 128
EPW = 5120
E_PAD = NW * EPW
NCH = EPW // CH
N_PAD = 10112
RPT = N_PAD // NS
DUMMY = N_NODES_C
DW = 128

ETILE = 2048
NTILE = 1264

_SQRT2 = math.sqrt(2.0)


def _dot(a, b):
    return lax.dot_general(a.astype(jnp.bfloat16), b.astype(jnp.bfloat16),
                           (((1,), (0,)), ((), ())),
                           preferred_element_type=jnp.float32)


def _gelu(x):
    return 0.5 * x * (1.0 + lax.erf(x / _SQRT2))


def _layernorm(x, g, b):
    m = jnp.mean(x, axis=1, keepdims=True)
    v = jnp.mean((x - m) * (x - m), axis=1, keepdims=True)
    return (x - m) * lax.rsqrt(v + 1e-5) * g + b


def _row2(v):
    return v.reshape(1, -1)


def _mlp2_body(x_ref, w1, b1, w2, b2, o_ref):
    h = _gelu(_dot(x_ref[...], w1[...]) + b1[...])
    o_ref[...] = _dot(h, w2[...]) + b2[...]


def _mlp2_call(x, p, tile):
    n, din = x.shape
    dh = p[0].shape[1]
    dout = p[2].shape[1]
    grid = (n // tile,)
    return pl.pallas_call(
        _mlp2_body,
        grid=grid,
        in_specs=[
            pl.BlockSpec((tile, din), lambda i: (i, 0)),
            pl.BlockSpec((din, dh), lambda i: (0, 0)),
            pl.BlockSpec((1, dh), lambda i: (0, 0)),
            pl.BlockSpec((dh, dout), lambda i: (0, 0)),
            pl.BlockSpec((1, dout), lambda i: (0, 0)),
        ],
        out_specs=pl.BlockSpec((tile, dout), lambda i: (i, 0)),
        out_shape=jax.ShapeDtypeStruct((n, dout), jnp.float32),
    )(x, p[0], _row2(p[1]), p[2], _row2(p[3]))


def _edge_block_body(sf_ref, df_ref, ef_ref, w1s, w1d, w1e, b1,
                     w2m, b2m, w2e, b2e, g_ref, bb_ref,
                     msg_ref, efo_ref):
    h = (_dot(sf_ref[...], w1s[...]) + _dot(df_ref[...], w1d[...])
         + _dot(ef_ref[...], w1e[...]) + b1[...])
    h = _gelu(h)
    msg_ref[...] = _dot(h[:, :HID], w2m[...]) + b2m[...]
    eu = _dot(h[:, HID:], w2e[...]) + b2e[...]
    efo_ref[...] = _layernorm(ef_ref[...] + eu, g_ref[...], bb_ref[...])


def _edge_block_call(sf, df, ef, wp):
    grid = (E_PAD // ETILE,)
    row = pl.BlockSpec((ETILE, HID), lambda i: (i, 0))
    cst = lambda s: pl.BlockSpec(s, lambda i: (0, 0))
    return pl.pallas_call(
        _edge_block_body,
        grid=grid,
        in_specs=[row, row, row,
                  cst((HID, 2 * HID)), cst((HID, 2 * HID)), cst((HID, 2 * HID)),
                  cst((1, 2 * HID)),
                  cst((HID, HID)), cst((1, HID)),
                  cst((HID, HID)), cst((1, HID)),
                  cst((1, HID)), cst((1, HID))],
        out_specs=[row, row],
        out_shape=[jax.ShapeDtypeStruct((E_PAD, HID), jnp.float32),
                   jax.ShapeDtypeStruct((E_PAD, HID), jnp.float32)],
    )(sf, df, ef, *wp)


def _node_block_body(nf_ref, a0, a1, d0, d1, w1a, w1b, b1, w2, b2, g_ref, bb_ref,
                     out_ref):
    deg = d0[:, :1] + d1[:, :1]
    agg = (a0[...] + a1[...]) / (deg + 1e-8)
    h = _gelu(_dot(nf_ref[...], w1a[...]) + _dot(agg, w1b[...]) + b1[...])
    nu = _dot(h, w2[...]) + b2[...]
    out_ref[...] = _layernorm(nf_ref[...] + nu, g_ref[...], bb_ref[...])


def _node_block_call(nf, a0, a1, d0, d1, wp):
    grid = (N_PAD // NTILE,)
    row = pl.BlockSpec((NTILE, HID), lambda i: (i, 0))
    drow = pl.BlockSpec((NTILE, DW), lambda i: (i, 0))
    cst = lambda s: pl.BlockSpec(s, lambda i: (0, 0))
    return pl.pallas_call(
        _node_block_body,
        grid=grid,
        in_specs=[row, row, row, drow, drow,
                  cst((HID, HID)), cst((HID, HID)), cst((1, HID)),
                  cst((HID, HID)), cst((1, HID)),
                  cst((1, HID)), cst((1, HID))],
        out_specs=row,
        out_shape=jax.ShapeDtypeStruct((N_PAD, HID), jnp.float32),
    )(nf, a0, a1, d0, d1, *wp)


def _softplus(x):
    return jnp.maximum(x, 0.0) + jnp.log(1.0 + jnp.exp(-jnp.abs(x)))


def _tail_body(nf_ref, wp, bp,
               w1_fe, b1_fe, w2_fe, b2_fe,
               w1_st, b1_st, w2_st, b2_st,
               w1_gp, b1_gp, w2_gp, b2_gp,
               w1_co, b1_co, w2_co, b2_co,
               w1_so, b1_so, w2_so, b2_so,
               o_fe, o_st, o_gp, o_co, o_so):
    pooled = _gelu(_dot(nf_ref[...], wp[...]) + bp[...])
    ridx = lax.broadcasted_iota(jnp.int32, (N_PAD, HID), 0)
    valid = ridx < N_NODES_C
    psum = jnp.sum(jnp.where(valid, pooled, 0.0), axis=0, keepdims=True)
    pmean = psum / float(N_NODES_C)
    pmax = jnp.max(jnp.where(valid, pooled, -1e30), axis=0, keepdims=True)
    g = jnp.concatenate([pmean, pmax], axis=1)

    def head(w1, b1, w2, b2):
        h = _gelu(_dot(g, w1[...]) + b1[...])
        return _dot(h, w2[...]) + b2[...]

    o_fe[...] = head(w1_fe, b1_fe, w2_fe, b2_fe)
    o_st[...] = jax.nn.sigmoid(head(w1_st, b1_st, w2_st, b2_st))
    o_gp[...] = _softplus(head(w1_gp, b1_gp, w2_gp, b2_gp))
    o_co[...] = head(w1_co, b1_co, w2_co, b2_co)
    o_so[...] = head(w1_so, b1_so, w2_so, b2_so)


def _tail_call(nf, pool, heads):
    cst = lambda s: pl.BlockSpec(s, lambda: (0, 0))
    args = [nf, pool[0], _row2(pool[1])]
    in_specs = [cst((N_PAD, HID)), cst((HID, HID)), cst((1, HID))]
    for name in ('fe', 'stab', 'gap', 'coord', 'solv'):
        w1, b1, w2, b2 = heads[name]
        k = w2.shape[1]
        w2p = jnp.pad(w2, ((0, 0), (0, 8 - k)))
        b2p = jnp.pad(b2, (0, 8 - k))
        args += [w1, _row2(b1), w2p, _row2(b2p)]
        in_specs += [cst((2 * HID, HID)), cst((1, HID)), cst((HID, 8)), cst((1, 8))]
    out_specs = [cst((1, 8))] * 5
    out_shape = [jax.ShapeDtypeStruct((1, 8), jnp.float32)] * 5
    return pl.pallas_call(
        _tail_body,
        in_specs=in_specs,
        out_specs=out_specs,
        out_shape=out_shape,
    )(*args)


def _sc_mesh():
    return plsc.VectorSubcoreMesh(core_axis_name="c", subcore_axis_name="s",
                                  num_cores=NC, num_subcores=NS)


@functools.cache
def _build_gather_kernel():
    return functools.partial(
        pl.kernel,
        out_type=(jax.ShapeDtypeStruct((E_PAD, HID), jnp.float32),
                  jax.ShapeDtypeStruct((E_PAD, HID), jnp.float32)),
        mesh=_sc_mesh(),
        scratch_types=[pltpu.VMEM((CH,), jnp.int32), pltpu.VMEM((CH,), jnp.int32),
                       pltpu.VMEM((CH,), jnp.int32), pltpu.VMEM((CH,), jnp.int32),
                       pltpu.VMEM((CH, HID), jnp.float32),
                       pltpu.VMEM((CH, HID), jnp.float32),
                       pltpu.VMEM((CH, HID), jnp.float32),
                       pltpu.VMEM((CH, HID), jnp.float32),
                       pltpu.SemaphoreType.DMA, pltpu.SemaphoreType.DMA,
                       pltpu.SemaphoreType.DMA, pltpu.SemaphoreType.DMA,
                       pltpu.SemaphoreType.DMA, pltpu.SemaphoreType.DMA,
                       pltpu.SemaphoreType.DMA, pltpu.SemaphoreType.DMA],
    )(_gather_body)


def _gather_body(nf_hbm, src_hbm, dst_hbm, sf_hbm, df_hbm,
                 si_a, di_a, si_b, di_b, sr_a, dr_a, sr_b, dr_b,
                 gs_a, gd_a, gs_b, gd_b, ws_a, wd_a, ws_b, wd_b):
    cid = lax.axis_index("c")
    sid = lax.axis_index("s")
    wbase = (cid * NS + sid) * EPW

    def wait_wb(rv, ws, wd):
        pltpu.make_async_copy(rv, sf_hbm.at[pl.ds(0, CH)], ws).wait()
        pltpu.make_async_copy(rv, df_hbm.at[pl.ds(0, CH)], wd).wait()

    def stage(base, si, di, sr, dr, gs, gd):
        pltpu.sync_copy(src_hbm.at[pl.ds(base, CH)], si)
        pltpu.sync_copy(dst_hbm.at[pl.ds(base, CH)], di)
        g1 = pltpu.async_copy(nf_hbm.at[si], sr, gs)
        g2 = pltpu.async_copy(nf_hbm.at[di], dr, gd)
        return g1, g2

    def flush(base, sr, dr, ws, wd, g1, g2):
        g1.wait()
        pltpu.async_copy(sr, sf_hbm.at[pl.ds(base, CH)], ws)
        g2.wait()
        pltpu.async_copy(dr, df_hbm.at[pl.ds(base, CH)], wd)

    def body(i, carry):
        base_a = pl.multiple_of(wbase + (2 * i) * CH, CH)
        base_b = pl.multiple_of(wbase + (2 * i + 1) * CH, CH)

        @pl.when(i > 0)
        def _():
            wait_wb(sr_a, ws_a, wd_a)

        ga = stage(base_a, si_a, di_a, sr_a, dr_a, gs_a, gd_a)

        @pl.when(i > 0)
        def _():
            wait_wb(sr_b, ws_b, wd_b)

        gb = stage(base_b, si_b, di_b, sr_b, dr_b, gs_b, gd_b)
        flush(base_a, sr_a, dr_a, ws_a, wd_a, *ga)
        flush(base_b, sr_b, dr_b, ws_b, wd_b, *gb)
        return carry

    lax.fori_loop(0, NCH // 2, body, 0)
    wait_wb(sr_a, ws_a, wd_a)
    wait_wb(sr_b, ws_b, wd_b)


@functools.cache
def _build_scatter_kernel():
    return functools.partial(
        pl.kernel,
        out_type=jax.ShapeDtypeStruct((NC, N_PAD, HID), jnp.float32),
        mesh=_sc_mesh(),
        scratch_types=[pltpu.VMEM((CH,), jnp.int32), pltpu.VMEM((CH,), jnp.int32),
                       pltpu.VMEM((CH, HID), jnp.float32),
                       pltpu.VMEM((CH, HID), jnp.float32),
                       pltpu.VMEM_SHARED((N_PAD, HID), jnp.float32),
                       pltpu.SemaphoreType.DMA, pltpu.SemaphoreType.DMA],
    )(_scatter_body)


def _scatter_body(msg_hbm, dst_hbm, zeros_hbm, out_hbm,
                  ix_a, ix_b, rw_a, rw_b, acc_sh, ls_a, ls_b):
    cid = lax.axis_index("c")
    sid = lax.axis_index("s")
    slab = pl.ds(sid * RPT, RPT)
    pltpu.sync_copy(zeros_hbm, acc_sh.at[slab])
    plsc.subcore_barrier()
    wbase = (cid * NS + sid) * EPW

    def fire(base, ix, rw, sem):
        pltpu.async_copy(dst_hbm.at[pl.ds(base, CH)], ix, sem)
        pltpu.async_copy(msg_hbm.at[pl.ds(base, CH)], rw, sem)

    def wait_load(ix, rw, sem):
        pltpu.make_async_copy(dst_hbm.at[pl.ds(0, CH)], ix, sem).wait()
        pltpu.make_async_copy(msg_hbm.at[pl.ds(0, CH)], rw, sem).wait()

    fire(pl.multiple_of(wbase, CH), ix_a, rw_a, ls_a)

    def body(i, carry):
        base_b = pl.multiple_of(wbase + (2 * i + 1) * CH, CH)
        wait_load(ix_a, rw_a, ls_a)
        fire(base_b, ix_b, rw_b, ls_b)
        pltpu.sync_copy(rw_a, acc_sh.at[ix_a], add=True)
        wait_load(ix_b, rw_b, ls_b)

        @pl.when(i < NCH // 2 - 1)
        def _():
            fire(pl.multiple_of(wbase + (2 * i + 2) * CH, CH), ix_a, rw_a, ls_a)

        pltpu.sync_copy(rw_b, acc_sh.at[ix_b], add=True)
        return carry

    lax.fori_loop(0, NCH // 2, body, 0)
    plsc.subcore_barrier()
    pltpu.sync_copy(acc_sh.at[slab], out_hbm.at[cid].at[slab])


@functools.cache
def _build_degree_kernel():
    return functools.partial(
        pl.kernel,
        out_type=jax.ShapeDtypeStruct((NC, N_PAD, DW), jnp.float32),
        mesh=_sc_mesh(),
        scratch_types=[pltpu.VMEM((CH,), jnp.int32),
                       pltpu.VMEM((CH, DW), jnp.float32),
                       pltpu.VMEM_SHARED((N_PAD, DW), jnp.float32)],
    )(_degree_body)


def _degree_body(dst_hbm, ones_hbm, zeros_hbm, out_hbm, idx_v, ones_v, acc_sh):
    cid = lax.axis_index("c")
    sid = lax.axis_index("s")
    slab = pl.ds(sid * RPT, RPT)
    pltpu.sync_copy(zeros_hbm, acc_sh.at[slab])
    pltpu.sync_copy(ones_hbm, ones_v)
    plsc.subcore_barrier()
    wbase = (cid * NS + sid) * EPW

    def body(c, carry):
        base = pl.multiple_of(wbase + c * CH, CH)
        pltpu.sync_copy(dst_hbm.at[pl.ds(base, CH)], idx_v)
        pltpu.sync_copy(ones_v, acc_sh.at[idx_v], add=True)
        return carry

    lax.fori_loop(0, NCH, body, 0)
    plsc.subcore_barrier()
    pltpu.sync_copy(acc_sh.at[slab], out_hbm.at[cid].at[slab])


def _edge_weights(blk):
    w1m, b1m, w2m, b2m = blk['msg']
    w1e, b1e, w2e, b2e = blk['edge_upd']
    w1s = jnp.concatenate([w1m[:HID], w1e[:HID]], axis=1)
    w1d = jnp.concatenate([w1m[HID:2 * HID], w1e[HID:2 * HID]], axis=1)
    w1f = jnp.concatenate([w1m[2 * HID:], w1e[2 * HID:]], axis=1)
    b1 = _row2(jnp.concatenate([b1m, b1e]))
    return (w1s, w1d, w1f, b1, w2m, _row2(b2m), w2e, _row2(b2e),
            _row2(blk['en_g']), _row2(blk['en_b']))


def _node_weights(blk):
    w1, b1, w2, b2 = blk['node_upd']
    return (w1[:HID], w1[HID:], _row2(b1), w2, _row2(b2),
            _row2(blk['nn_g']), _row2(blk['nn_b']))


def kernel(node_feat_raw, edge_feat_raw, edge_index, params):
    n_nodes = node_feat_raw.shape[0]
    n_edges = edge_feat_raw.shape[0]

    nfr = jnp.pad(node_feat_raw, ((0, N_PAD - n_nodes), (0, 0)))
    efr = jnp.pad(edge_feat_raw, ((0, E_PAD - n_edges), (0, 0)))
    src = jnp.pad(edge_index[0], (0, E_PAD - n_edges))
    dst = jnp.pad(edge_index[1], (0, E_PAD - n_edges),
                  constant_values=DUMMY)

    zeros_big = jnp.zeros((RPT, HID), jnp.float32)
    zeros_deg = jnp.zeros((RPT, DW), jnp.float32)
    ones_deg = jnp.ones((CH, DW), jnp.float32)

    nf = _mlp2_call(nfr, params['node_emb'], NTILE)
    ef = _mlp2_call(efr, params['edge_emb'], ETILE)

    degs = _build_degree_kernel()(dst, ones_deg, zeros_deg)
    d0 = degs[0]
    d1 = degs[1]

    for blk in params['blocks']:
        sf, df = _build_gather_kernel()(nf, src, dst)
        msg, ef = _edge_block_call(sf, df, ef, _edge_weights(blk))
        aggs = _build_scatter_kernel()(msg, dst, zeros_big)
        nf = _node_block_call(nf, aggs[0], aggs[1], d0, d1, _node_weights(blk))

    o_fe, o_st, o_gp, o_co, o_so = _tail_call(nf, params['pool'], params['heads'])
    return jnp.concatenate([o_fe[0, :1], o_st[0, :1], o_gp[0, :1],
                            o_co[0, :4], o_so[0, :3]])


# speedup vs baseline: 1.1713x; 1.0138x over previous
import functools
import math

import jax
import jax.numpy as jnp
from jax import lax
from jax.experimental import pallas as pl
from jax.experimental.pallas import tpu as pltpu
from jax.experimental.pallas import tpu_sc as plsc

HID = 128
N_NODES_C = 10000
N_EDGES_C = 160000

NC, NS = 2, 16
NW = NC * NS
CH = 128
EPW = 5120
E_PAD = NW * EPW
NCH = EPW // CH
N_PAD = 10112
RPT = N_PAD // NS
DUMMY = N_NODES_C
DW = 128

ETILE = 2048
NTILE = 1264

_SQRT2 = math.sqrt(2.0)


def _dot(a, b):
    return lax.dot_general(a.astype(jnp.bfloat16), b.astype(jnp.bfloat16),
                           (((1,), (0,)), ((), ())),
                           preferred_element_type=jnp.float32)


def _gelu(x):
    return 0.5 * x * (1.0 + lax.erf(x / _SQRT2))


def _layernorm(x, g, b):
    m = jnp.mean(x, axis=1, keepdims=True)
    v = jnp.mean((x - m) * (x - m), axis=1, keepdims=True)
    return (x - m) * lax.rsqrt(v + 1e-5) * g + b


def _row2(v):
    return v.reshape(1, -1)


def _mlp2_body(x_ref, w1, b1, w2, b2, o_ref):
    h = _gelu(_dot(x_ref[...], w1[...]) + b1[...])
    o_ref[...] = _dot(h, w2[...]) + b2[...]


def _mlp2_call(x, p, tile):
    n, din = x.shape
    dh = p[0].shape[1]
    dout = p[2].shape[1]
    grid = (n // tile,)
    return pl.pallas_call(
        _mlp2_body,
        grid=grid,
        in_specs=[
            pl.BlockSpec((tile, din), lambda i: (i, 0)),
            pl.BlockSpec((din, dh), lambda i: (0, 0)),
            pl.BlockSpec((1, dh), lambda i: (0, 0)),
            pl.BlockSpec((dh, dout), lambda i: (0, 0)),
            pl.BlockSpec((1, dout), lambda i: (0, 0)),
        ],
        out_specs=pl.BlockSpec((tile, dout), lambda i: (i, 0)),
        out_shape=jax.ShapeDtypeStruct((n, dout), jnp.float32),
    )(x, p[0], _row2(p[1]), p[2], _row2(p[3]))


def _edge_block_body(sf_ref, df_ref, ef_ref, w1s, w1d, w1e, b1,
                     w2m, b2m, w2e, b2e, g_ref, bb_ref,
                     msg_ref, efo_ref):
    h = (_dot(sf_ref[...], w1s[...]) + _dot(df_ref[...], w1d[...])
         + _dot(ef_ref[...], w1e[...]) + b1[...])
    h = _gelu(h)
    msg_ref[...] = _dot(h[:, :HID], w2m[...]) + b2m[...]
    eu = _dot(h[:, HID:], w2e[...]) + b2e[...]
    efo_ref[...] = _layernorm(ef_ref[...] + eu, g_ref[...], bb_ref[...])


def _edge_block_call(sf, df, ef, wp):
    grid = (E_PAD // ETILE,)
    row = pl.BlockSpec((ETILE, HID), lambda i: (i, 0))
    cst = lambda s: pl.BlockSpec(s, lambda i: (0, 0))
    return pl.pallas_call(
        _edge_block_body,
        grid=grid,
        in_specs=[row, row, row,
                  cst((HID, 2 * HID)), cst((HID, 2 * HID)), cst((HID, 2 * HID)),
                  cst((1, 2 * HID)),
                  cst((HID, HID)), cst((1, HID)),
                  cst((HID, HID)), cst((1, HID)),
                  cst((1, HID)), cst((1, HID))],
        out_specs=[row, row],
        out_shape=[jax.ShapeDtypeStruct((E_PAD, HID), jnp.float32),
                   jax.ShapeDtypeStruct((E_PAD, HID), jnp.float32)],
    )(sf, df, ef, *wp)


def _node_block_body(nf_ref, a0, a1, d0, d1, w1a, w1b, b1, w2, b2, g_ref, bb_ref,
                     out_ref):
    deg = d0[:, :1] + d1[:, :1]
    agg = (a0[...] + a1[...]) / (deg + 1e-8)
    h = _gelu(_dot(nf_ref[...], w1a[...]) + _dot(agg, w1b[...]) + b1[...])
    nu = _dot(h, w2[...]) + b2[...]
    out_ref[...] = _layernorm(nf_ref[...] + nu, g_ref[...], bb_ref[...])


def _node_block_call(nf, a0, a1, d0, d1, wp):
    grid = (N_PAD // NTILE,)
    row = pl.BlockSpec((NTILE, HID), lambda i: (i, 0))
    drow = pl.BlockSpec((NTILE, DW), lambda i: (i, 0))
    cst = lambda s: pl.BlockSpec(s, lambda i: (0, 0))
    return pl.pallas_call(
        _node_block_body,
        grid=grid,
        in_specs=[row, row, row, drow, drow,
                  cst((HID, HID)), cst((HID, HID)), cst((1, HID)),
                  cst((HID, HID)), cst((1, HID)),
                  cst((1, HID)), cst((1, HID))],
        out_specs=row,
        out_shape=jax.ShapeDtypeStruct((N_PAD, HID), jnp.float32),
    )(nf, a0, a1, d0, d1, *wp)


def _softplus(x):
    return jnp.maximum(x, 0.0) + jnp.log(1.0 + jnp.exp(-jnp.abs(x)))


def _tail_body(nf_ref, wp, bp,
               w1_fe, b1_fe, w2_fe, b2_fe,
               w1_st, b1_st, w2_st, b2_st,
               w1_gp, b1_gp, w2_gp, b2_gp,
               w1_co, b1_co, w2_co, b2_co,
               w1_so, b1_so, w2_so, b2_so,
               o_fe, o_st, o_gp, o_co, o_so):
    pooled = _gelu(_dot(nf_ref[...], wp[...]) + bp[...])
    ridx = lax.broadcasted_iota(jnp.int32, (N_PAD, HID), 0)
    valid = ridx < N_NODES_C
    psum = jnp.sum(jnp.where(valid, pooled, 0.0), axis=0, keepdims=True)
    pmean = psum / float(N_NODES_C)
    pmax = jnp.max(jnp.where(valid, pooled, -1e30), axis=0, keepdims=True)
    g = jnp.concatenate([pmean, pmax], axis=1)

    def head(w1, b1, w2, b2):
        h = _gelu(_dot(g, w1[...]) + b1[...])
        return _dot(h, w2[...]) + b2[...]

    o_fe[...] = head(w1_fe, b1_fe, w2_fe, b2_fe)
    o_st[...] = jax.nn.sigmoid(head(w1_st, b1_st, w2_st, b2_st))
    o_gp[...] = _softplus(head(w1_gp, b1_gp, w2_gp, b2_gp))
    o_co[...] = head(w1_co, b1_co, w2_co, b2_co)
    o_so[...] = head(w1_so, b1_so, w2_so, b2_so)


def _tail_call(nf, pool, heads):
    cst = lambda s: pl.BlockSpec(s, lambda: (0, 0))
    args = [nf, pool[0], _row2(pool[1])]
    in_specs = [cst((N_PAD, HID)), cst((HID, HID)), cst((1, HID))]
    for name in ('fe', 'stab', 'gap', 'coord', 'solv'):
        w1, b1, w2, b2 = heads[name]
        k = w2.shape[1]
        w2p = jnp.pad(w2, ((0, 0), (0, 8 - k)))
        b2p = jnp.pad(b2, (0, 8 - k))
        args += [w1, _row2(b1), w2p, _row2(b2p)]
        in_specs += [cst((2 * HID, HID)), cst((1, HID)), cst((HID, 8)), cst((1, 8))]
    out_specs = [cst((1, 8))] * 5
    out_shape = [jax.ShapeDtypeStruct((1, 8), jnp.float32)] * 5
    return pl.pallas_call(
        _tail_body,
        in_specs=in_specs,
        out_specs=out_specs,
        out_shape=out_shape,
    )(*args)


def _sc_mesh():
    return plsc.VectorSubcoreMesh(core_axis_name="c", subcore_axis_name="s",
                                  num_cores=NC, num_subcores=NS)


@functools.cache
def _build_gather_kernel():
    return functools.partial(
        pl.kernel,
        out_type=(jax.ShapeDtypeStruct((E_PAD, HID), jnp.float32),
                  jax.ShapeDtypeStruct((E_PAD, HID), jnp.float32)),
        mesh=_sc_mesh(),
        scratch_types=[pltpu.VMEM((EPW,), jnp.int32), pltpu.VMEM((EPW,), jnp.int32),
                       pltpu.VMEM((CH, HID), jnp.float32),
                       pltpu.VMEM((CH, HID), jnp.float32),
                       pltpu.VMEM((CH, HID), jnp.float32),
                       pltpu.VMEM((CH, HID), jnp.float32),
                       pltpu.SemaphoreType.DMA, pltpu.SemaphoreType.DMA,
                       pltpu.SemaphoreType.DMA, pltpu.SemaphoreType.DMA,
                       pltpu.SemaphoreType.DMA, pltpu.SemaphoreType.DMA,
                       pltpu.SemaphoreType.DMA, pltpu.SemaphoreType.DMA],
    )(_gather_body)


def _gather_body(nf_hbm, src_hbm, dst_hbm, sf_hbm, df_hbm,
                 si_all, di_all, sr_a, dr_a, sr_b, dr_b,
                 gs_a, gd_a, gs_b, gd_b, ws_a, wd_a, ws_b, wd_b):
    cid = lax.axis_index("c")
    sid = lax.axis_index("s")
    wbase = (cid * NS + sid) * EPW
    pltpu.sync_copy(src_hbm.at[pl.ds(pl.multiple_of(wbase, CH), EPW)], si_all)
    pltpu.sync_copy(dst_hbm.at[pl.ds(pl.multiple_of(wbase, CH), EPW)], di_all)

    def wait_wb(rv, ws, wd):
        pltpu.make_async_copy(rv, sf_hbm.at[pl.ds(0, CH)], ws).wait()
        pltpu.make_async_copy(rv, df_hbm.at[pl.ds(0, CH)], wd).wait()

    def stage(off, sr, dr, gs, gd):
        g1 = pltpu.async_copy(nf_hbm.at[si_all.at[pl.ds(off, CH)]], sr, gs)
        g2 = pltpu.async_copy(nf_hbm.at[di_all.at[pl.ds(off, CH)]], dr, gd)
        return g1, g2

    def flush(base, sr, dr, ws, wd, g1, g2):
        g1.wait()
        pltpu.async_copy(sr, sf_hbm.at[pl.ds(base, CH)], ws)
        g2.wait()
        pltpu.async_copy(dr, df_hbm.at[pl.ds(base, CH)], wd)

    def body(i, carry):
        off_a = pl.multiple_of((2 * i) * CH, CH)
        off_b = pl.multiple_of((2 * i + 1) * CH, CH)
        base_a = pl.multiple_of(wbase + (2 * i) * CH, CH)
        base_b = pl.multiple_of(wbase + (2 * i + 1) * CH, CH)

        @pl.when(i > 0)
        def _():
            wait_wb(sr_a, ws_a, wd_a)

        ga = stage(off_a, sr_a, dr_a, gs_a, gd_a)

        @pl.when(i > 0)
        def _():
            wait_wb(sr_b, ws_b, wd_b)

        gb = stage(off_b, sr_b, dr_b, gs_b, gd_b)
        flush(base_a, sr_a, dr_a, ws_a, wd_a, *ga)
        flush(base_b, sr_b, dr_b, ws_b, wd_b, *gb)
        return carry

    lax.fori_loop(0, NCH // 2, body, 0)
    wait_wb(sr_a, ws_a, wd_a)
    wait_wb(sr_b, ws_b, wd_b)


@functools.cache
def _build_scatter_kernel():
    return functools.partial(
        pl.kernel,
        out_type=jax.ShapeDtypeStruct((NC, N_PAD, HID), jnp.float32),
        mesh=_sc_mesh(),
        scratch_types=[pltpu.VMEM((CH,), jnp.int32), pltpu.VMEM((CH,), jnp.int32),
                       pltpu.VMEM((CH, HID), jnp.float32),
                       pltpu.VMEM((CH, HID), jnp.float32),
                       pltpu.VMEM_SHARED((N_PAD, HID), jnp.float32),
                       pltpu.SemaphoreType.DMA, pltpu.SemaphoreType.DMA],
    )(_scatter_body)


def _scatter_body(msg_hbm, dst_hbm, zeros_hbm, out_hbm,
                  ix_a, ix_b, rw_a, rw_b, acc_sh, ls_a, ls_b):
    cid = lax.axis_index("c")
    sid = lax.axis_index("s")
    slab = pl.ds(sid * RPT, RPT)
    pltpu.sync_copy(zeros_hbm, acc_sh.at[slab])
    plsc.subcore_barrier()
    wbase = (cid * NS + sid) * EPW

    def fire(base, ix, rw, sem):
        pltpu.async_copy(dst_hbm.at[pl.ds(base, CH)], ix, sem)
        pltpu.async_copy(msg_hbm.at[pl.ds(base, CH)], rw, sem)

    def wait_load(ix, rw, sem):
        pltpu.make_async_copy(dst_hbm.at[pl.ds(0, CH)], ix, sem).wait()
        pltpu.make_async_copy(msg_hbm.at[pl.ds(0, CH)], rw, sem).wait()

    fire(pl.multiple_of(wbase, CH), ix_a, rw_a, ls_a)

    def body(i, carry):
        base_b = pl.multiple_of(wbase + (2 * i + 1) * CH, CH)
        wait_load(ix_a, rw_a, ls_a)
        fire(base_b, ix_b, rw_b, ls_b)
        pltpu.sync_copy(rw_a, acc_sh.at[ix_a], add=True)
        wait_load(ix_b, rw_b, ls_b)

        @pl.when(i < NCH // 2 - 1)
        def _():
            fire(pl.multiple_of(wbase + (2 * i + 2) * CH, CH), ix_a, rw_a, ls_a)

        pltpu.sync_copy(rw_b, acc_sh.at[ix_b], add=True)
        return carry

    lax.fori_loop(0, NCH // 2, body, 0)
    plsc.subcore_barrier()
    pltpu.sync_copy(acc_sh.at[slab], out_hbm.at[cid].at[slab])


@functools.cache
def _build_degree_kernel():
    return functools.partial(
        pl.kernel,
        out_type=jax.ShapeDtypeStruct((NC, N_PAD, DW), jnp.float32),
        mesh=_sc_mesh(),
        scratch_types=[pltpu.VMEM((CH,), jnp.int32),
                       pltpu.VMEM((CH, DW), jnp.float32),
                       pltpu.VMEM_SHARED((N_PAD, DW), jnp.float32)],
    )(_degree_body)


def _degree_body(dst_hbm, ones_hbm, zeros_hbm, out_hbm, idx_v, ones_v, acc_sh):
    cid = lax.axis_index("c")
    sid = lax.axis_index("s")
    slab = pl.ds(sid * RPT, RPT)
    pltpu.sync_copy(zeros_hbm, acc_sh.at[slab])
    pltpu.sync_copy(ones_hbm, ones_v)
    plsc.subcore_barrier()
    wbase = (cid * NS + sid) * EPW

    def body(c, carry):
        base = pl.multiple_of(wbase + c * CH, CH)
        pltpu.sync_copy(dst_hbm.at[pl.ds(base, CH)], idx_v)
        pltpu.sync_copy(ones_v, acc_sh.at[idx_v], add=True)
        return carry

    lax.fori_loop(0, NCH, body, 0)
    plsc.subcore_barrier()
    pltpu.sync_copy(acc_sh.at[slab], out_hbm.at[cid].at[slab])


def _edge_weights(blk):
    w1m, b1m, w2m, b2m = blk['msg']
    w1e, b1e, w2e, b2e = blk['edge_upd']
    w1s = jnp.concatenate([w1m[:HID], w1e[:HID]], axis=1)
    w1d = jnp.concatenate([w1m[HID:2 * HID], w1e[HID:2 * HID]], axis=1)
    w1f = jnp.concatenate([w1m[2 * HID:], w1e[2 * HID:]], axis=1)
    b1 = _row2(jnp.concatenate([b1m, b1e]))
    return (w1s, w1d, w1f, b1, w2m, _row2(b2m), w2e, _row2(b2e),
            _row2(blk['en_g']), _row2(blk['en_b']))


def _node_weights(blk):
    w1, b1, w2, b2 = blk['node_upd']
    return (w1[:HID], w1[HID:], _row2(b1), w2, _row2(b2),
            _row2(blk['nn_g']), _row2(blk['nn_b']))


def kernel(node_feat_raw, edge_feat_raw, edge_index, params):
    n_nodes = node_feat_raw.shape[0]
    n_edges = edge_feat_raw.shape[0]

    nfr = jnp.pad(node_feat_raw, ((0, N_PAD - n_nodes), (0, 0)))
    efr = jnp.pad(edge_feat_raw, ((0, E_PAD - n_edges), (0, 0)))
    src = jnp.pad(edge_index[0], (0, E_PAD - n_edges))
    dst = jnp.pad(edge_index[1], (0, E_PAD - n_edges),
                  constant_values=DUMMY)

    zeros_big = jnp.zeros((RPT, HID), jnp.float32)
    zeros_deg = jnp.zeros((RPT, DW), jnp.float32)
    ones_deg = jnp.ones((CH, DW), jnp.float32)

    nf = _mlp2_call(nfr, params['node_emb'], NTILE)
    ef = _mlp2_call(efr, params['edge_emb'], ETILE)

    degs = _build_degree_kernel()(dst, ones_deg, zeros_deg)
    d0 = degs[0]
    d1 = degs[1]

    for blk in params['blocks']:
        sf, df = _build_gather_kernel()(nf, src, dst)
        msg, ef = _edge_block_call(sf, df, ef, _edge_weights(blk))
        aggs = _build_scatter_kernel()(msg, dst, zeros_big)
        nf = _node_block_call(nf, aggs[0], aggs[1], d0, d1, _node_weights(blk))

    o_fe, o_st, o_gp, o_co, o_so = _tail_call(nf, params['pool'], params['heads'])
    return jnp.concatenate([o_fe[0, :1], o_st[0, :1], o_gp[0, :1],
                            o_co[0, :4], o_so[0, :3]])
